# baseline (speedup 1.0000x reference)
import jax
import jax.numpy as jnp
from jax import lax
from jax.experimental import pallas as pl
from jax.experimental.pallas import tpu as pltpu, tpu_sc as plsc

NUM_ITEMS = 1000000
NUM_WORKERS = 1000
BATCH = 16384

NC = 2
NS = 16
NW = NC * NS
BPW = BATCH // NW
LANES = 16


def _tile_body(win_hbm, los_hbm, ann_hbm, item_hbm, beta_hbm, out_hbm,
               win_v, los_v, ann_v, rw_v, rl_v, beta_v, out_v,
               sem_in, sem_g):
    wid = lax.axis_index("s") * NC + lax.axis_index("c")

    cp_w = pltpu.async_copy(win_hbm.at[wid], win_v, sem_in)
    cp_l = pltpu.async_copy(los_hbm.at[wid], los_v, sem_in)
    cp_a = pltpu.async_copy(ann_hbm.at[wid], ann_v, sem_in)
    cp_b = pltpu.async_copy(beta_hbm, beta_v, sem_in)
    cp_w.wait()
    cp_l.wait()

    g1 = pltpu.async_copy(item_hbm.at[win_v.at[0]], rw_v, sem_g)
    g2 = pltpu.async_copy(item_hbm.at[los_v.at[0]], rl_v, sem_g)
    cp_a.wait()
    cp_b.wait()
    g1.wait()
    g2.wait()

    ann_f = ann_v.at[0]

    @pl.loop(0, BPW // LANES, unroll=4)
    def _compute(i):
        sl = pl.ds(i * LANES, LANES)
        bt = plsc.load_gather(beta_v, [ann_f[sl]])
        out_v[sl] = bt * (rw_v[sl] - rl_v[sl])

    pltpu.sync_copy(out_v, out_hbm.at[pl.ds(wid * BPW, BPW)])


ITEM_PAD = 1000448
BETA_PAD = 1024


def kernel(winners, losers, annotators, item_rewards, worker_betas):
    win = winners.astype(jnp.int32).reshape(NW, 1, BPW)
    los = losers.astype(jnp.int32).reshape(NW, 1, BPW)
    ann = annotators.astype(jnp.int32).reshape(NW, 1, BPW)
    item = jnp.pad(item_rewards.T,
                   ((0, 0), (0, ITEM_PAD - NUM_ITEMS))).reshape(ITEM_PAD)
    beta = worker_betas.reshape(NUM_WORKERS)

    mesh = plsc.VectorSubcoreMesh(core_axis_name="c", subcore_axis_name="s",
                                  num_cores=NC, num_subcores=NS)
    out = pl.kernel(
        _tile_body,
        out_type=jax.ShapeDtypeStruct((BATCH,), jnp.float32),
        mesh=mesh,
        scratch_types=[
            pltpu.VMEM((1, BPW), jnp.int32),
            pltpu.VMEM((1, BPW), jnp.int32),
            pltpu.VMEM((1, BPW), jnp.int32),
            pltpu.VMEM((BPW,), jnp.float32),
            pltpu.VMEM((BPW,), jnp.float32),
            pltpu.VMEM((NUM_WORKERS,), jnp.float32),
            pltpu.VMEM((BPW,), jnp.float32),
            pltpu.SemaphoreType.DMA,
            pltpu.SemaphoreType.DMA,
        ],
        compiler_params=pltpu.CompilerParams(needs_layout_passes=False),
    )(win, los, ann, item, beta)
    return out


# speedup vs baseline: 1.0050x; 1.0050x over previous
import jax
import jax.numpy as jnp
from jax import lax
from jax.experimental import pallas as pl
from jax.experimental.pallas import tpu as pltpu, tpu_sc as plsc

NUM_ITEMS = 1000000
NUM_WORKERS = 1000
BATCH = 16384

NC = 2
NS = 16
NW = NC * NS
BPW = BATCH // NW
LANES = 16


def _tile_body(win_hbm, los_hbm, ann_hbm, item_hbm, beta_hbm, out_hbm,
               win_v, los_v, ann_v, rw_v, rl_v, beta_v, out_v,
               sem_in, sem_g):
    wid = lax.axis_index("s") * NC + lax.axis_index("c")

    cp_w = pltpu.async_copy(win_hbm.at[wid], win_v, sem_in)
    cp_l = pltpu.async_copy(los_hbm.at[wid], los_v, sem_in)
    cp_a = pltpu.async_copy(ann_hbm.at[wid], ann_v, sem_in)
    cp_b = pltpu.async_copy(beta_hbm, beta_v, sem_in)
    cp_w.wait()
    cp_l.wait()

    HALF = BPW // 2
    win_f = win_v.at[0]
    los_f = los_v.at[0]
    lo, hi = pl.ds(0, HALF), pl.ds(HALF, HALF)
    g1a = pltpu.async_copy(item_hbm.at[win_f.at[lo]], rw_v.at[lo], sem_g)
    g2a = pltpu.async_copy(item_hbm.at[los_f.at[lo]], rl_v.at[lo], sem_g)
    g1b = pltpu.async_copy(item_hbm.at[win_f.at[hi]], rw_v.at[hi], sem_g)
    g2b = pltpu.async_copy(item_hbm.at[los_f.at[hi]], rl_v.at[hi], sem_g)
    cp_a.wait()
    cp_b.wait()

    ann_f = ann_v.at[0]

    def compute_range(base):
        @pl.loop(0, HALF // LANES, unroll=4)
        def _compute(i):
            sl = pl.ds(base + i * LANES, LANES)
            bt = plsc.load_gather(beta_v, [ann_f[sl]])
            out_v[sl] = bt * (rw_v[sl] - rl_v[sl])

    g1a.wait()
    g2a.wait()
    compute_range(0)
    g1b.wait()
    g2b.wait()
    compute_range(HALF)

    pltpu.sync_copy(out_v, out_hbm.at[pl.ds(wid * BPW, BPW)])


ITEM_PAD = 1000448
BETA_PAD = 1024


def kernel(winners, losers, annotators, item_rewards, worker_betas):
    win = winners.astype(jnp.int32).reshape(NW, 1, BPW)
    los = losers.astype(jnp.int32).reshape(NW, 1, BPW)
    ann = annotators.astype(jnp.int32).reshape(NW, 1, BPW)
    item = jnp.pad(item_rewards.T,
                   ((0, 0), (0, ITEM_PAD - NUM_ITEMS))).reshape(ITEM_PAD)
    beta = worker_betas.reshape(NUM_WORKERS)

    mesh = plsc.VectorSubcoreMesh(core_axis_name="c", subcore_axis_name="s",
                                  num_cores=NC, num_subcores=NS)
    out = pl.kernel(
        _tile_body,
        out_type=jax.ShapeDtypeStruct((BATCH,), jnp.float32),
        mesh=mesh,
        scratch_types=[
            pltpu.VMEM((1, BPW), jnp.int32),
            pltpu.VMEM((1, BPW), jnp.int32),
            pltpu.VMEM((1, BPW), jnp.int32),
            pltpu.VMEM((BPW,), jnp.float32),
            pltpu.VMEM((BPW,), jnp.float32),
            pltpu.VMEM((NUM_WORKERS,), jnp.float32),
            pltpu.VMEM((BPW,), jnp.float32),
            pltpu.SemaphoreType.DMA,
            pltpu.SemaphoreType.DMA,
        ],
        compiler_params=pltpu.CompilerParams(needs_layout_passes=False),
    )(win, los, ann, item, beta)
    return out
